# Initial kernel scaffold; baseline (speedup 1.0000x reference)
import jax
import jax.numpy as jnp
from jax.experimental import pallas as pl


def kernel(x, edge_index, edge_attr, batch, We_imp, Wn_imp, We_g, Wn_g, W1, b1, W2, b2):
    raise NotImplementedError("write your pallas kernel here")


# speedup vs baseline: 2.9627x; 2.9627x over previous
import jax
import jax.numpy as jnp
from jax import lax
from jax.experimental import pallas as pl
from jax.experimental.pallas import tpu as pltpu
from jax.experimental.pallas import tpu_sc as plsc

N = 10000
E = 320000
D = 128
B = 128

TPS = 16
C = 128
K = 160
KG = 8
G = K // KG
EPT = K * C
E_PAD = TPS * EPT
NPAD = 10240
RPT = NPAD // TPS
DE = 4

R = 512
NB = NPAD // R


def _sc_body(x_hbm, src_hbm, dst_hbm, ea_hbm, zt_hbm,
             t_hbm, a_hbm, acc_sh, rows_v, src_v, dst_v, gsem):
    c = lax.axis_index("c")
    s = lax.axis_index("s")
    row0 = s * RPT

    pltpu.sync_copy(zt_hbm.at[pl.ds(row0, RPT)], acc_sh.at[pl.ds(row0, RPT)])

    plsc.subcore_barrier()

    def group(g, carry):
        pltpu.sync_copy(src_hbm.at[s, pl.ds(g * KG, KG)], src_v)
        pltpu.sync_copy(dst_hbm.at[s, pl.ds(g * KG, KG)], dst_v)

        def chunk(j, carry2):
            @pl.when(c == 0)
            def _():
                pltpu.async_copy(x_hbm.at[src_v.at[j]], rows_v, gsem).wait()

            @pl.when(c == 1)
            def _():
                pltpu.sync_copy(ea_hbm.at[s, g * KG + j], rows_v)

            pltpu.sync_copy(rows_v, acc_sh.at[dst_v.at[j]], add=True)
            return carry2

        return lax.fori_loop(0, KG, chunk, carry)

    lax.fori_loop(0, G, group, 0)
    plsc.subcore_barrier()

    @pl.when(c == 0)
    def _():
        pltpu.sync_copy(acc_sh.at[pl.ds(row0, RPT)], t_hbm.at[pl.ds(row0, RPT)])

    @pl.when(c == 1)
    def _():
        pltpu.sync_copy(acc_sh.at[pl.ds(row0, RPT)], a_hbm.at[pl.ds(row0, RPT)])


def _sc_scatter():
  return pl.kernel(
    _sc_body,
    out_type=[
        jax.ShapeDtypeStruct((NPAD, D), jnp.float32),
        jax.ShapeDtypeStruct((NPAD, D), jnp.float32),
    ],
    mesh=plsc.VectorSubcoreMesh(core_axis_name="c", subcore_axis_name="s",
                                num_cores=2, num_subcores=TPS),
    scratch_types=[
        pltpu.VMEM_SHARED((NPAD, D), jnp.float32),
        pltpu.VMEM((C, D), jnp.float32),
        pltpu.VMEM((KG, C), jnp.int32),
        pltpu.VMEM((KG, C), jnp.int32),
        pltpu.SemaphoreType.DMA,
    ],
  )


def _tc_body(x_r, t_r, a_r, brow_r, bcol_r,
             wei_r, wni_r, weg_r, wng_r, w1_r, b1_r, w2_r, b2_r,
             z_r, seg, cnt, pooled):
    g = pl.program_id(0)
    f32 = jnp.float32
    NEG = jnp.float32(-3.0e38)

    @pl.when(g == 0)
    def _():
        seg[...] = jnp.full((1, B), NEG, f32)
        cnt[...] = jnp.zeros((B, 1), f32)
        pooled[...] = jnp.zeros((B, D), f32)

    @pl.when(g < 2 * NB)
    def _():
        xs = x_r[...] + t_r[...]
        a = a_r[...]
        ni_full = (xs + a @ wei_r[...]) @ wni_r[...]
        ni = jnp.sum(ni_full, axis=1, keepdims=True)
        bcol = bcol_r[...]
        m2 = jnp.broadcast_to(bcol, (R, B)) == lax.broadcasted_iota(
            jnp.int32, (R, B), 1)
        m2f = m2.astype(f32)

        @pl.when(g < NB)
        def _():
            cand = jnp.where(m2, jnp.broadcast_to(ni, (R, B)), NEG)
            seg[...] = jnp.maximum(seg[...], jnp.max(cand, axis=0, keepdims=True))
            brow = brow_r[...]
            m1 = jnp.broadcast_to(brow, (B, R)) == lax.broadcasted_iota(
                jnp.int32, (B, R), 0)
            cnt[...] += jnp.sum(m1.astype(f32), axis=1, keepdims=True)

        @pl.when(g >= NB)
        def _():
            segrow = jnp.broadcast_to(seg[...], (R, B))
            outc = jnp.sum(m2f * segrow, axis=1, keepdims=True)
            outc = outc + (1.0 - jnp.sum(m2f, axis=1, keepdims=True))
            imp = ni / (outc * 10.0) + 0.9
            h = jnp.maximum((xs + a @ weg_r[...]) @ wng_r[...], 0.0)
            hi = h * imp
            brow = brow_r[...]
            m1 = jnp.broadcast_to(brow, (B, R)) == lax.broadcasted_iota(
                jnp.int32, (B, R), 0)
            pooled[...] += jnp.dot(m1.astype(f32), hi,
                                   preferred_element_type=f32)

    @pl.when(g == 2 * NB)
    def _():
        pm = pooled[...] / jnp.maximum(cnt[...], 1.0)
        z1 = jnp.maximum(pm @ w1_r[...] + b1_r[...], 0.0)
        z_r[...] = z1 @ w2_r[...] + b2_r[...]


def _tc_call(xp, t, a, brow, bcol, wei, wni, weg, wng, w1, b1r, w2, b2r):
    blk = lambda g: (lax.rem(g, NB), 0)
    full = lambda g: (0, 0)
    return pl.pallas_call(
        _tc_body,
        grid=(2 * NB + 1,),
        in_specs=[
            pl.BlockSpec((R, D), blk),
            pl.BlockSpec((R, D), blk),
            pl.BlockSpec((R, D), blk),
            pl.BlockSpec((1, R), lambda g: (0, lax.rem(g, NB))),
            pl.BlockSpec((R, 1), blk),
            pl.BlockSpec((D, D), full),
            pl.BlockSpec((D, D), full),
            pl.BlockSpec((D, D), full),
            pl.BlockSpec((D, D), full),
            pl.BlockSpec((D, D), full),
            pl.BlockSpec((1, D), full),
            pl.BlockSpec((D, D), full),
            pl.BlockSpec((1, D), full),
        ],
        out_specs=pl.BlockSpec((B, D), full),
        out_shape=jax.ShapeDtypeStruct((B, D), jnp.float32),
        scratch_shapes=[
            pltpu.VMEM((1, B), jnp.float32),
            pltpu.VMEM((B, 1), jnp.float32),
            pltpu.VMEM((B, D), jnp.float32),
        ],
        compiler_params=pltpu.CompilerParams(
            dimension_semantics=("arbitrary",)),
    )(xp, t, a, brow, bcol, wei, wni, weg, wng, w1, b1r, w2, b2r)


def kernel(x, edge_index, edge_attr, batch, We_imp, Wn_imp, We_g, Wn_g,
           W1, b1, W2, b2):
    pad_e = E_PAD - E
    src = jnp.concatenate([edge_index[0], jnp.zeros((pad_e,), jnp.int32)])
    dst = jnp.concatenate([edge_index[1],
                           jnp.full((pad_e,), N, jnp.int32)])
    srcp = src.reshape(TPS, K, C)
    dstp = dst.reshape(TPS, K, C)
    eap = jnp.pad(edge_attr, ((0, pad_e), (0, D - DE)))
    eap = eap.reshape(TPS, K, C, D)
    zt = jnp.zeros((NPAD, D), jnp.float32)

    t, aw = _sc_scatter()(x, srcp, dstp, eap, zt)

    xp = jnp.pad(x, ((0, NPAD - N), (0, 0)))
    bpad = jnp.pad(batch, (0, NPAD - N), constant_values=B)
    brow = bpad.reshape(1, NPAD)
    bcol = bpad.reshape(NPAD, 1)
    wei = jnp.pad(We_imp, ((0, D - We_imp.shape[0]), (0, 0)))
    weg = jnp.pad(We_g, ((0, D - We_g.shape[0]), (0, 0)))
    wni = jnp.pad(Wn_imp, ((0, 0), (0, D - Wn_imp.shape[1])))
    b1r = b1.reshape(1, D)
    b2r = b2.reshape(1, D)

    return _tc_call(xp, t, aw, brow, bcol, wei, wni, weg, wng=Wn_g,
                    w1=W1, b1r=b1r, w2=W2, b2r=b2r)
